# Initial kernel scaffold; baseline (speedup 1.0000x reference)
import jax
import jax.numpy as jnp
from jax.experimental import pallas as pl


def kernel(x, edge_index, enc_W, enc_b, ls_W, ls_b, ln_W, ln_b, dec_W, dec_b):
    raise NotImplementedError("write your pallas kernel here")


# speedup vs baseline: 2.8735x; 2.8735x over previous
import functools

import jax
import jax.numpy as jnp
from jax import lax
from jax.experimental import pallas as pl
from jax.experimental.pallas import tpu as pltpu
from jax.experimental.pallas import tpu_sc as plsc

N = 10000
E = 320000
D = 128
NC = 2
NS = 16
NW = NC * NS
C = 128
CH = 80
EPW = CH * C
E_PAD = NW * EPW
AGG_ROWS = 10240
ROWS_PER_W = AGG_ROWS // NS
PAD_DST = N + 8

RB = 1000
GRID = N // RB


def _gelu(x):
    return 0.5 * x * (1.0 + lax.erf(x * 0.7071067811865476))


def _segment_sum_sc(h, src_r, dst_r, zeros):
    mesh = plsc.VectorSubcoreMesh(
        core_axis_name="core", subcore_axis_name="subcore")

    @functools.partial(
        pl.kernel,
        out_type=jax.ShapeDtypeStruct((NC, AGG_ROWS, D), jnp.float32),
        mesh=mesh,
        scratch_types=[
            pltpu.VMEM((CH, C), jnp.int32),
            pltpu.VMEM((CH, C), jnp.int32),
            pltpu.VMEM((C, D), jnp.float32),
            pltpu.VMEM_SHARED((AGG_ROWS, D), jnp.float32),
        ],
    )
    def seg_kernel(h_hbm, src_hbm, dst_hbm, z_hbm, out_hbm,
                   srcv, dstv, rows, agg_sh):
        c = lax.axis_index("core")
        s = lax.axis_index("subcore")
        w = c * NS + s
        pltpu.sync_copy(z_hbm.at[pl.ds(s * ROWS_PER_W, ROWS_PER_W)],
                        agg_sh.at[pl.ds(s * ROWS_PER_W, ROWS_PER_W)])
        pltpu.sync_copy(src_hbm.at[w], srcv)
        pltpu.sync_copy(dst_hbm.at[w], dstv)
        plsc.subcore_barrier()

        @pl.loop(0, CH)
        def _(j):
            pltpu.sync_copy(h_hbm.at[srcv.at[j]], rows)
            pltpu.sync_copy(rows, agg_sh.at[dstv.at[j]], add=True)

        plsc.subcore_barrier()
        pltpu.sync_copy(agg_sh.at[pl.ds(s * ROWS_PER_W, ROWS_PER_W)],
                        out_hbm.at[c, pl.ds(s * ROWS_PER_W, ROWS_PER_W)])

    return seg_kernel(h, src_r, dst_r, zeros)


def _enc_tc(x, Wt, b):
    def body(x_ref, w_ref, b_ref, o_ref):
        o_ref[...] = _gelu(
            jnp.dot(x_ref[...], w_ref[...],
                    preferred_element_type=jnp.float32) + b_ref[...])

    return pl.pallas_call(
        body,
        grid=(GRID,),
        in_specs=[
            pl.BlockSpec((RB, D), lambda i: (i, 0)),
            pl.BlockSpec((D, D), lambda i: (0, 0)),
            pl.BlockSpec((1, D), lambda i: (0, 0)),
        ],
        out_specs=pl.BlockSpec((RB, D), lambda i: (i, 0)),
        out_shape=jax.ShapeDtypeStruct((N, D), jnp.float32),
    )(x, Wt, b)


def _layer_tc(h, aggp, lsWt, lsb, lnWt, lnb):
    def body(h_ref, a_ref, lsw_ref, lsb_ref, lnw_ref, lnb_ref, o_ref):
        agg = a_ref[0] + a_ref[1]
        msg = jnp.dot(agg, lnw_ref[...],
                      preferred_element_type=jnp.float32) + lnb_ref[...]
        hs = jnp.dot(h_ref[...], lsw_ref[...],
                     preferred_element_type=jnp.float32) + lsb_ref[...]
        o_ref[...] = _gelu(hs + msg) + h_ref[...]

    return pl.pallas_call(
        body,
        grid=(GRID,),
        in_specs=[
            pl.BlockSpec((RB, D), lambda i: (i, 0)),
            pl.BlockSpec((NC, RB, D), lambda i: (0, i, 0)),
            pl.BlockSpec((D, D), lambda i: (0, 0)),
            pl.BlockSpec((1, D), lambda i: (0, 0)),
            pl.BlockSpec((D, D), lambda i: (0, 0)),
            pl.BlockSpec((1, D), lambda i: (0, 0)),
        ],
        out_specs=pl.BlockSpec((RB, D), lambda i: (i, 0)),
        out_shape=jax.ShapeDtypeStruct((N, D), jnp.float32),
    )(h, aggp, lsWt, lsb, lnWt, lnb)


def _dec_tc(h, Wt, b):
    def body(h_ref, w_ref, b_ref, o_ref):
        o_ref[...] = jnp.dot(h_ref[...], w_ref[...],
                             preferred_element_type=jnp.float32) + b_ref[...]

    return pl.pallas_call(
        body,
        grid=(GRID,),
        in_specs=[
            pl.BlockSpec((RB, D), lambda i: (i, 0)),
            pl.BlockSpec((D, D), lambda i: (0, 0)),
            pl.BlockSpec((1, D), lambda i: (0, 0)),
        ],
        out_specs=pl.BlockSpec((RB, D), lambda i: (i, 0)),
        out_shape=jax.ShapeDtypeStruct((N, D), jnp.float32),
    )(h, Wt, b)


def kernel(x, edge_index, enc_W, enc_b, ls_W, ls_b, ln_W, ln_b, dec_W, dec_b):
    pad = E_PAD - E
    src_r = jnp.concatenate(
        [edge_index[0], jnp.zeros((pad,), jnp.int32)]).reshape(NW, CH, C)
    dst_r = jnp.concatenate(
        [edge_index[1], jnp.full((pad,), PAD_DST, jnp.int32)]).reshape(NW, CH, C)
    zeros = jnp.zeros((AGG_ROWS, D), jnp.float32)

    h = _enc_tc(x, enc_W.T, enc_b.reshape(1, D))
    for k in range(2):
        aggp = _segment_sum_sc(h, src_r, dst_r, zeros)
        h = _layer_tc(h, aggp, ls_W[k].T, ls_b[k].reshape(1, D),
                      ln_W[k].T, ln_b[k].reshape(1, D))
    return _dec_tc(h, dec_W.T, dec_b.reshape(1, D))
